# baseline (speedup 1.0000x reference)
import functools

import jax
import jax.numpy as jnp
from jax import lax
from jax.experimental import pallas as pl
from jax.experimental.pallas import tpu as pltpu
from jax.experimental.pallas import tpu_sc as plsc

B, N, C = 4, 8192, 64
S = 2048
NS0, NS1 = 16, 32
R0SQ, R1SQ = 0.1 * 0.1, 0.2 * 0.2
OUT = 64
F = C + 3 + 1
K = 32
NIDX = NS0 + NS1
TS = 64
NBLK = S // TS
TW = 128
NTILE = N // TW
EB = 2

NC_SC, NS_SC, LANES = 2, 16, 16
NW = NC_SC * NS_SC
CPW = (B * S) // NW
CH = 2
NCHUNK = CPW // CH
GI = CH * NIDX


def _topk_body(posT_ref, snew_ref, idx_ref, d2_ref, vals_ref, idxs_ref):
    b = pl.program_id(0)
    posT = posT_ref[0]
    snew = snew_ref[0]
    dot = lax.dot_general(snew, posT, (((1,), (0,)), ((), ())),
                          preferred_element_type=jnp.float32)
    pn = jnp.sum(posT * posT, axis=0, keepdims=True)
    sn = jnp.sum(snew * snew, axis=1, keepdims=True)
    d2_ref[...] = sn + pn - 2.0 * dot

    iota_f = lax.broadcasted_iota(jnp.int32, (TS, N), 1)
    iota_l = lax.broadcasted_iota(jnp.int32, (TS, TW), 1)
    inf = jnp.float32(jnp.inf)

    def super_body(sb, carry):
        ams = []
        for e in range(EB):
            mv = jnp.full((TS, TW), inf, jnp.float32)
            mt = jnp.full((TS, TW), NTILE, jnp.int32)
            for jt in range(NTILE):
                t = d2_ref[:, jt * TW:(jt + 1) * TW]
                for am in ams:
                    t = jnp.where(iota_l + jt * TW == am, inf, t)
                upd = t < mv
                mt = jnp.where(upd, jt, mt)
                mv = jnp.minimum(mv, t)
            rmin = jnp.min(mv, axis=1, keepdims=True)
            gidx = mt * TW + iota_l
            am = jnp.min(jnp.where(mv == rmin, gidx, N), axis=1,
                         keepdims=True)
            kpos = sb * EB + e
            vals_ref[pl.ds(kpos, 1), :] = jnp.reshape(rmin, (1, TS))
            idxs_ref[pl.ds(kpos, 1), :] = jnp.reshape(am, (1, TS))
            ams.append(am)
        d2 = d2_ref[...]
        pred = iota_f == ams[0]
        for am in ams[1:]:
            pred = pred | (iota_f == am)
        d2_ref[...] = jnp.where(pred, inf, d2)
        return carry

    lax.fori_loop(0, K // EB, super_body, 0)

    vals = vals_ref[...]
    idxs = idxs_ref[...]
    base = idxs[0:1, :]
    e0 = jnp.where(vals[:NS0] <= R0SQ, idxs[:NS0], base)
    e1 = jnp.where(vals <= R1SQ, idxs, base)
    idx_ref[0, 0] = jnp.concatenate([e0, e1], axis=0) + b * N


def _ball_query(posT, new_pos):
    return pl.pallas_call(
        _topk_body,
        grid=(B, NBLK),
        in_specs=[
            pl.BlockSpec((1, 3, N), lambda b, s: (b, 0, 0)),
            pl.BlockSpec((1, TS, 3), lambda b, s: (b, s, 0)),
        ],
        out_specs=pl.BlockSpec((1, 1, NIDX, TS), lambda b, s: (b, s, 0, 0)),
        out_shape=jax.ShapeDtypeStruct((B, NBLK, NIDX, TS), jnp.int32),
        scratch_shapes=[
            pltpu.VMEM((TS, N), jnp.float32),
            pltpu.VMEM((K, TS), jnp.float32),
            pltpu.VMEM((K, TS), jnp.int32),
        ],
    )(posT, new_pos)


def _mm_body(a_ref, w_ref, o_ref):
    ab = jnp.dot(a_ref[...], w_ref[...], preferred_element_type=jnp.float32)
    o_ref[...] = jnp.concatenate(
        [ab, jnp.zeros((TM, 128 - OUT), jnp.float32)], axis=1)


TM = 1024


def _proj(a_all, w_all):
    rows = a_all.shape[0]
    return pl.pallas_call(
        _mm_body,
        grid=(rows // TM,),
        in_specs=[
            pl.BlockSpec((TM, F), lambda i: (i, 0)),
            pl.BlockSpec((F, OUT), lambda i: (0, 0)),
        ],
        out_specs=pl.BlockSpec((TM, 128), lambda i: (i, 0)),
        out_shape=jax.ShapeDtypeStruct((rows, 128), jnp.float32),
    )(a_all, w_all)


def _sc_gather_max(proj, idxf):
    mesh = plsc.VectorSubcoreMesh(core_axis_name="c", subcore_axis_name="s")

    @functools.partial(
        pl.kernel,
        mesh=mesh,
        out_type=jax.ShapeDtypeStruct((B * S, 2 * OUT), jnp.float32),
        scratch_types=[
            pltpu.VMEM((GI,), jnp.int32),
            pltpu.VMEM((GI, 128), jnp.float32),
            pltpu.VMEM((CH, 128), jnp.float32),
            pltpu.VMEM((CH, 2 * OUT), jnp.float32),
            pltpu.SemaphoreType.DMA,
        ],
    )
    def k(p_hbm, idx_hbm, out_hbm, idx_v, rows_v, c_v, o_v, sem):
        wid = lax.axis_index("c") * NS_SC + lax.axis_index("s")
        base0 = wid * CPW

        def chunk(i, carry):
            cb = base0 + i * CH
            pltpu.sync_copy(idx_hbm.at[pl.ds(cb * NIDX, GI)], idx_v)
            pltpu.async_copy(p_hbm.at[idx_v], rows_v, sem).wait()
            pltpu.sync_copy(p_hbm.at[pl.ds(B * N + cb, CH)], c_v)
            for cc in range(CH):
                r0 = cc * NIDX
                for col in range(OUT // LANES):
                    sl = pl.ds(col * LANES, LANES)
                    cvec = c_v[cc, sl]
                    m0 = rows_v[r0, sl]
                    for j in range(1, NS0):
                        m0 = jnp.maximum(m0, rows_v[r0 + j, sl])
                    o_v[cc, sl] = jnp.maximum(m0 - cvec, 0.0)
                    m1 = rows_v[r0 + NS0, sl]
                    for j in range(NS0 + 1, NIDX):
                        m1 = jnp.maximum(m1, rows_v[r0 + j, sl])
                    o_v[cc, pl.ds(OUT + col * LANES, LANES)] = (
                        jnp.maximum(m1 - cvec, 0.0))
            pltpu.sync_copy(o_v, out_hbm.at[pl.ds(cb, CH)])
            return carry

        lax.fori_loop(0, NCHUNK, chunk, 0)

    return k(proj, idxf)


def kernel(x, pos, W, b):
    idxc = jnp.arange(S, dtype=jnp.int32) * (N // S)
    new_pos = pos[:, idxc, :]
    posT = jnp.transpose(pos, (0, 2, 1))

    idx48 = _ball_query(posT, new_pos)
    idxf = jnp.transpose(idx48, (0, 1, 3, 2)).reshape(-1)

    zg = jnp.zeros((B, N, 1), jnp.float32)
    a_g = jnp.concatenate([pos, x, zg], axis=-1).reshape(B * N, F)
    zc = jnp.zeros((B, S, C), jnp.float32)
    oc = jnp.ones((B, S, 1), jnp.float32)
    a_c = jnp.concatenate([new_pos, zc, oc], axis=-1).reshape(B * S, F)
    a_all = jnp.concatenate([a_g, a_c], axis=0)
    w_all = jnp.concatenate([W, -b[None, :]], axis=0)

    proj = _proj(a_all, w_all)
    out = _sc_gather_max(proj, idxf)
    return out.reshape(B, S, 2 * OUT), new_pos


# speedup vs baseline: 1.1888x; 1.1888x over previous
import functools

import jax
import jax.numpy as jnp
from jax import lax
from jax.experimental import pallas as pl
from jax.experimental.pallas import tpu as pltpu
from jax.experimental.pallas import tpu_sc as plsc

B, N, C = 4, 8192, 64
S = 2048
NS0, NS1 = 16, 32
R0SQ, R1SQ = 0.1 * 0.1, 0.2 * 0.2
OUT = 64
F = C + 3 + 1
K = 32
NIDX = NS0 + NS1
TS = 256
NBLK = S // TS

NC_SC, NS_SC, LANES = 2, 16, 16
NW = NC_SC * NS_SC
CPW = (B * S) // NW
CH = 4
NCHUNK = CPW // CH
GI = CH * NIDX
GH = GI // 2


def _topk_body(posT_ref, snew_ref, idx_ref, d2_ref, vals_ref, idxs_ref):
    b = pl.program_id(0)
    posT = posT_ref[0]
    snew = snew_ref[0]
    dot = lax.dot_general(snew, posT, (((1,), (0,)), ((), ())),
                          preferred_element_type=jnp.float32)
    pn = jnp.sum(posT * posT, axis=0, keepdims=True)
    sn = jnp.sum(snew * snew, axis=1, keepdims=True)
    d2_ref[...] = sn + pn - 2.0 * dot

    iota = lax.broadcasted_iota(jnp.int32, (TS, N), 1)

    def body(k, carry):
        d2 = d2_ref[...]
        m = jnp.min(d2, axis=1, keepdims=True)
        am = jnp.min(jnp.where(d2 == m, iota, N), axis=1,
                     keepdims=True)
        d2_ref[...] = jnp.where(iota == am, jnp.float32(jnp.inf), d2)
        vals_ref[pl.ds(k, 1), :] = jnp.reshape(m, (1, TS))
        idxs_ref[pl.ds(k, 1), :] = jnp.reshape(am, (1, TS))
        return carry

    lax.fori_loop(0, K, body, 0)

    vals = vals_ref[...]
    idxs = idxs_ref[...]
    base = idxs[0:1, :]
    e0 = jnp.where(vals[:NS0] <= R0SQ, idxs[:NS0], base)
    e1 = jnp.where(vals <= R1SQ, idxs, base)
    idx_ref[0, 0] = jnp.concatenate([e0, e1], axis=0) + b * N


def _ball_query(posT, new_pos):
    return pl.pallas_call(
        _topk_body,
        grid=(B, NBLK),
        in_specs=[
            pl.BlockSpec((1, 3, N), lambda b, s: (b, 0, 0)),
            pl.BlockSpec((1, TS, 3), lambda b, s: (b, s, 0)),
        ],
        out_specs=pl.BlockSpec((1, 1, NIDX, TS), lambda b, s: (b, s, 0, 0)),
        out_shape=jax.ShapeDtypeStruct((B, NBLK, NIDX, TS), jnp.int32),
        scratch_shapes=[
            pltpu.VMEM((TS, N), jnp.float32),
            pltpu.VMEM((K, TS), jnp.float32),
            pltpu.VMEM((K, TS), jnp.int32),
        ],
    )(posT, new_pos)


def _mm_body(a_ref, w_ref, o_ref):
    ab = jnp.dot(a_ref[...], w_ref[...], preferred_element_type=jnp.float32)
    o_ref[...] = jnp.concatenate(
        [ab, jnp.zeros((TM, 128 - OUT), jnp.float32)], axis=1)


TM = 1024


def _proj(a_all, w_all):
    rows = a_all.shape[0]
    return pl.pallas_call(
        _mm_body,
        grid=(rows // TM,),
        in_specs=[
            pl.BlockSpec((TM, F), lambda i: (i, 0)),
            pl.BlockSpec((F, OUT), lambda i: (0, 0)),
        ],
        out_specs=pl.BlockSpec((TM, 128), lambda i: (i, 0)),
        out_shape=jax.ShapeDtypeStruct((rows, 128), jnp.float32),
    )(a_all, w_all)


def _sc_gather_max(proj, idxf):
    mesh = plsc.VectorSubcoreMesh(core_axis_name="c", subcore_axis_name="s")

    @functools.partial(
        pl.kernel,
        mesh=mesh,
        out_type=jax.ShapeDtypeStruct((B * S, 2 * OUT), jnp.float32),
        scratch_types=[
            pltpu.VMEM((GI,), jnp.int32),
            pltpu.VMEM((GI,), jnp.int32),
            pltpu.VMEM((GI, 128), jnp.float32),
            pltpu.VMEM((GI, 128), jnp.float32),
            pltpu.VMEM((CH, 128), jnp.float32),
            pltpu.VMEM((CH, 128), jnp.float32),
            pltpu.VMEM((CH, 2 * OUT), jnp.float32),
            pltpu.SemaphoreType.DMA,
            pltpu.SemaphoreType.DMA,
        ],
    )
    def k(p_hbm, idx_hbm, out_hbm, idx_v0, idx_v1, rows_v0, rows_v1,
          c_v0, c_v1, o_v, sem0, sem1):
        wid = lax.axis_index("c") * NS_SC + lax.axis_index("s")
        base0 = wid * CPW
        sems = (sem0, sem1)
        idxs_v = (idx_v0, idx_v1)
        rows_vs = (rows_v0, rows_v1)
        c_vs = (c_v0, c_v1)

        def fire(ci, buf):
            cb = base0 + ci * CH
            sem = sems[buf]
            idx_v, rows_v, c_v = idxs_v[buf], rows_vs[buf], c_vs[buf]
            pltpu.sync_copy(idx_hbm.at[pl.ds(cb * NIDX, GI)], idx_v)
            pltpu.async_copy(p_hbm.at[idx_v.at[pl.ds(0, GH)]],
                             rows_v.at[pl.ds(0, GH)], sem)
            pltpu.async_copy(p_hbm.at[idx_v.at[pl.ds(GH, GH)]],
                             rows_v.at[pl.ds(GH, GH)], sem)
            pltpu.async_copy(p_hbm.at[pl.ds(B * N + cb, CH)], c_v, sem)

        def drain(buf):
            sem = sems[buf]
            pltpu.make_async_copy(p_hbm.at[pl.ds(0, GI)],
                                  rows_vs[buf], sem).wait()
            pltpu.make_async_copy(p_hbm.at[pl.ds(0, CH)],
                                  c_vs[buf], sem).wait()

        def compute(ci, buf):
            cb = base0 + ci * CH
            rows_v, c_v = rows_vs[buf], c_vs[buf]
            for cc in range(CH):
                r0 = cc * NIDX
                for col in range(OUT // LANES):
                    sl = pl.ds(col * LANES, LANES)
                    cvec = c_v[cc, sl]
                    m0 = rows_v[r0, sl]
                    for j in range(1, NS0):
                        m0 = jnp.maximum(m0, rows_v[r0 + j, sl])
                    o_v[cc, sl] = jnp.maximum(m0 - cvec, 0.0)
                    m1 = rows_v[r0 + NS0, sl]
                    for j in range(NS0 + 1, NIDX):
                        m1 = jnp.maximum(m1, rows_v[r0 + j, sl])
                    o_v[cc, pl.ds(OUT + col * LANES, LANES)] = (
                        jnp.maximum(m1 - cvec, 0.0))
            pltpu.sync_copy(o_v, out_hbm.at[pl.ds(cb, CH)])

        fire(0, 0)

        def body2(j, carry):
            i0 = 2 * j
            fire(i0 + 1, 1)
            drain(0)
            compute(i0, 0)

            @pl.when(i0 + 2 < NCHUNK)
            def _():
                fire(i0 + 2, 0)

            drain(1)
            compute(i0 + 1, 1)
            return carry

        lax.fori_loop(0, NCHUNK // 2, body2, 0)

    return k(proj, idxf)


def kernel(x, pos, W, b):
    idxc = jnp.arange(S, dtype=jnp.int32) * (N // S)
    new_pos = pos[:, idxc, :]
    posT = jnp.transpose(pos, (0, 2, 1))

    idx48 = _ball_query(posT, new_pos)
    idxf = jnp.transpose(idx48, (0, 1, 3, 2)).reshape(-1)

    zg = jnp.zeros((B, N, 1), jnp.float32)
    a_g = jnp.concatenate([pos, x, zg], axis=-1).reshape(B * N, F)
    zc = jnp.zeros((B, S, C), jnp.float32)
    oc = jnp.ones((B, S, 1), jnp.float32)
    a_c = jnp.concatenate([new_pos, zc, oc], axis=-1).reshape(B * S, F)
    a_all = jnp.concatenate([a_g, a_c], axis=0)
    w_all = jnp.concatenate([W, -b[None, :]], axis=0)

    proj = _proj(a_all, w_all)
    out = _sc_gather_max(proj, idxf)
    return out.reshape(B, S, 2 * OUT), new_pos


# speedup vs baseline: 1.1965x; 1.0065x over previous
import functools

import jax
import jax.numpy as jnp
from jax import lax
from jax.experimental import pallas as pl
from jax.experimental.pallas import tpu as pltpu
from jax.experimental.pallas import tpu_sc as plsc

B, N, C = 4, 8192, 64
S = 2048
NS0, NS1 = 16, 32
R0SQ, R1SQ = 0.1 * 0.1, 0.2 * 0.2
OUT = 64
F = C + 3 + 1
K = 32
NIDX = NS0 + NS1
TS = 256
NBLK = S // TS

NC_SC, NS_SC, LANES = 2, 16, 16
NW = NC_SC * NS_SC
CPW = (B * S) // NW
CH = 4
NCHUNK = CPW // CH
GI = CH * NIDX
GH = GI // 2


def _topk_body(posT_ref, snew_ref, idx_ref, d2_ref, vals_ref, idxs_ref):
    b = pl.program_id(0)
    posT = posT_ref[0]
    snew = snew_ref[0]
    dot = lax.dot_general(snew, posT, (((1,), (0,)), ((), ())),
                          preferred_element_type=jnp.float32)
    pn = jnp.sum(posT * posT, axis=0, keepdims=True)
    sn = jnp.sum(snew * snew, axis=1, keepdims=True)
    d2_ref[...] = sn + pn - 2.0 * dot

    iota = lax.broadcasted_iota(jnp.int32, (TS, N), 1)
    inf = jnp.float32(jnp.inf)

    def body2(j, carry):
        d2a = d2_ref[...]
        m0 = jnp.min(d2a, axis=1, keepdims=True)
        am0 = jnp.min(jnp.where(d2a == m0, iota, N), axis=1,
                      keepdims=True)
        excl0 = iota == am0
        d2b = jnp.where(excl0, inf, d2a)
        m1 = jnp.min(d2b, axis=1, keepdims=True)
        am1 = jnp.min(jnp.where(d2b == m1, iota, N), axis=1,
                      keepdims=True)
        d2_ref[...] = jnp.where(excl0 | (iota == am1), inf, d2a)
        vals_ref[pl.ds(2 * j, 1), :] = jnp.reshape(m0, (1, TS))
        idxs_ref[pl.ds(2 * j, 1), :] = jnp.reshape(am0, (1, TS))
        vals_ref[pl.ds(2 * j + 1, 1), :] = jnp.reshape(m1, (1, TS))
        idxs_ref[pl.ds(2 * j + 1, 1), :] = jnp.reshape(am1, (1, TS))
        return carry

    lax.fori_loop(0, K // 2, body2, 0)

    vals = vals_ref[...]
    idxs = idxs_ref[...]
    base = idxs[0:1, :]
    e0 = jnp.where(vals[:NS0] <= R0SQ, idxs[:NS0], base)
    e1 = jnp.where(vals <= R1SQ, idxs, base)
    idx_ref[0, 0] = jnp.concatenate([e0, e1], axis=0) + b * N


def _ball_query(posT, new_pos):
    return pl.pallas_call(
        _topk_body,
        grid=(B, NBLK),
        in_specs=[
            pl.BlockSpec((1, 3, N), lambda b, s: (b, 0, 0)),
            pl.BlockSpec((1, TS, 3), lambda b, s: (b, s, 0)),
        ],
        out_specs=pl.BlockSpec((1, 1, NIDX, TS), lambda b, s: (b, s, 0, 0)),
        out_shape=jax.ShapeDtypeStruct((B, NBLK, NIDX, TS), jnp.int32),
        scratch_shapes=[
            pltpu.VMEM((TS, N), jnp.float32),
            pltpu.VMEM((K, TS), jnp.float32),
            pltpu.VMEM((K, TS), jnp.int32),
        ],
    )(posT, new_pos)


def _mm_body(a_ref, w_ref, o_ref):
    ab = jnp.dot(a_ref[...], w_ref[...], preferred_element_type=jnp.float32)
    o_ref[...] = jnp.concatenate(
        [ab, jnp.zeros((TM, 128 - OUT), jnp.float32)], axis=1)


TM = 1024


def _proj(a_all, w_all):
    rows = a_all.shape[0]
    return pl.pallas_call(
        _mm_body,
        grid=(rows // TM,),
        in_specs=[
            pl.BlockSpec((TM, F), lambda i: (i, 0)),
            pl.BlockSpec((F, OUT), lambda i: (0, 0)),
        ],
        out_specs=pl.BlockSpec((TM, 128), lambda i: (i, 0)),
        out_shape=jax.ShapeDtypeStruct((rows, 128), jnp.float32),
    )(a_all, w_all)


def _sc_gather_max(proj, idxf):
    mesh = plsc.VectorSubcoreMesh(core_axis_name="c", subcore_axis_name="s")

    @functools.partial(
        pl.kernel,
        mesh=mesh,
        out_type=jax.ShapeDtypeStruct((B * S, 2 * OUT), jnp.float32),
        scratch_types=[
            pltpu.VMEM((GI,), jnp.int32),
            pltpu.VMEM((GI,), jnp.int32),
            pltpu.VMEM((GI, 128), jnp.float32),
            pltpu.VMEM((GI, 128), jnp.float32),
            pltpu.VMEM((CH, 128), jnp.float32),
            pltpu.VMEM((CH, 128), jnp.float32),
            pltpu.VMEM((CH, 2 * OUT), jnp.float32),
            pltpu.SemaphoreType.DMA,
            pltpu.SemaphoreType.DMA,
        ],
    )
    def k(p_hbm, idx_hbm, out_hbm, idx_v0, idx_v1, rows_v0, rows_v1,
          c_v0, c_v1, o_v, sem0, sem1):
        wid = lax.axis_index("c") * NS_SC + lax.axis_index("s")
        base0 = wid * CPW
        sems = (sem0, sem1)
        idxs_v = (idx_v0, idx_v1)
        rows_vs = (rows_v0, rows_v1)
        c_vs = (c_v0, c_v1)

        def fire(ci, buf):
            cb = base0 + ci * CH
            sem = sems[buf]
            idx_v, rows_v, c_v = idxs_v[buf], rows_vs[buf], c_vs[buf]
            pltpu.sync_copy(idx_hbm.at[pl.ds(cb * NIDX, GI)], idx_v)
            pltpu.async_copy(p_hbm.at[idx_v.at[pl.ds(0, GH)]],
                             rows_v.at[pl.ds(0, GH)], sem)
            pltpu.async_copy(p_hbm.at[idx_v.at[pl.ds(GH, GH)]],
                             rows_v.at[pl.ds(GH, GH)], sem)
            pltpu.async_copy(p_hbm.at[pl.ds(B * N + cb, CH)], c_v, sem)

        def drain(buf):
            sem = sems[buf]
            pltpu.make_async_copy(p_hbm.at[pl.ds(0, GI)],
                                  rows_vs[buf], sem).wait()
            pltpu.make_async_copy(p_hbm.at[pl.ds(0, CH)],
                                  c_vs[buf], sem).wait()

        def compute(ci, buf):
            cb = base0 + ci * CH
            rows_v, c_v = rows_vs[buf], c_vs[buf]
            for cc in range(CH):
                r0 = cc * NIDX
                for col in range(OUT // LANES):
                    sl = pl.ds(col * LANES, LANES)
                    cvec = c_v[cc, sl]
                    m0 = rows_v[r0, sl]
                    for j in range(1, NS0):
                        m0 = jnp.maximum(m0, rows_v[r0 + j, sl])
                    o_v[cc, sl] = jnp.maximum(m0 - cvec, 0.0)
                    m1 = rows_v[r0 + NS0, sl]
                    for j in range(NS0 + 1, NIDX):
                        m1 = jnp.maximum(m1, rows_v[r0 + j, sl])
                    o_v[cc, pl.ds(OUT + col * LANES, LANES)] = (
                        jnp.maximum(m1 - cvec, 0.0))
            pltpu.sync_copy(o_v, out_hbm.at[pl.ds(cb, CH)])

        fire(0, 0)

        def body2(j, carry):
            i0 = 2 * j
            fire(i0 + 1, 1)
            drain(0)
            compute(i0, 0)

            @pl.when(i0 + 2 < NCHUNK)
            def _():
                fire(i0 + 2, 0)

            drain(1)
            compute(i0 + 1, 1)
            return carry

        lax.fori_loop(0, NCHUNK // 2, body2, 0)

    return k(proj, idxf)


def kernel(x, pos, W, b):
    idxc = jnp.arange(S, dtype=jnp.int32) * (N // S)
    new_pos = pos[:, idxc, :]
    posT = jnp.transpose(pos, (0, 2, 1))

    idx48 = _ball_query(posT, new_pos)
    idxf = jnp.transpose(idx48, (0, 1, 3, 2)).reshape(-1)

    zg = jnp.zeros((B, N, 1), jnp.float32)
    a_g = jnp.concatenate([pos, x, zg], axis=-1).reshape(B * N, F)
    zc = jnp.zeros((B, S, C), jnp.float32)
    oc = jnp.ones((B, S, 1), jnp.float32)
    a_c = jnp.concatenate([new_pos, zc, oc], axis=-1).reshape(B * S, F)
    a_all = jnp.concatenate([a_g, a_c], axis=0)
    w_all = jnp.concatenate([W, -b[None, :]], axis=0)

    proj = _proj(a_all, w_all)
    out = _sc_gather_max(proj, idxf)
    return out.reshape(B, S, 2 * OUT), new_pos


# speedup vs baseline: 1.2165x; 1.0167x over previous
import functools

import jax
import jax.numpy as jnp
from jax import lax
from jax.experimental import pallas as pl
from jax.experimental.pallas import tpu as pltpu
from jax.experimental.pallas import tpu_sc as plsc

B, N, C = 4, 8192, 64
S = 2048
NS0, NS1 = 16, 32
R0SQ, R1SQ = 0.1 * 0.1, 0.2 * 0.2
OUT = 64
F = C + 3 + 1
K = 32
NIDX = NS0 + NS1
TS = 512
NBLK = S // TS

NC_SC, NS_SC, LANES = 2, 16, 16
NW = NC_SC * NS_SC
CPW = (B * S) // NW
CH = 4
NCHUNK = CPW // CH
GI = CH * NIDX
GH = GI // 2


def _topk_body(posT_ref, snew_ref, idx_ref, d2_ref, vals_ref, idxs_ref):
    b = pl.program_id(0)
    posT = posT_ref[0]
    snew = snew_ref[0]
    dot = lax.dot_general(snew, posT, (((1,), (0,)), ((), ())),
                          preferred_element_type=jnp.float32)
    pn = jnp.sum(posT * posT, axis=0, keepdims=True)
    sn = jnp.sum(snew * snew, axis=1, keepdims=True)
    d2_ref[...] = sn + pn - 2.0 * dot

    iota = lax.broadcasted_iota(jnp.int32, (TS, N), 1)
    inf = jnp.float32(jnp.inf)

    def body2(j, carry):
        d2a = d2_ref[...]
        m0 = jnp.min(d2a, axis=1, keepdims=True)
        am0 = jnp.min(jnp.where(d2a == m0, iota, N), axis=1,
                      keepdims=True)
        excl0 = iota == am0
        d2b = jnp.where(excl0, inf, d2a)
        m1 = jnp.min(d2b, axis=1, keepdims=True)
        am1 = jnp.min(jnp.where(d2b == m1, iota, N), axis=1,
                      keepdims=True)
        d2_ref[...] = jnp.where(excl0 | (iota == am1), inf, d2a)
        vals_ref[pl.ds(2 * j, 1), :] = jnp.reshape(m0, (1, TS))
        idxs_ref[pl.ds(2 * j, 1), :] = jnp.reshape(am0, (1, TS))
        vals_ref[pl.ds(2 * j + 1, 1), :] = jnp.reshape(m1, (1, TS))
        idxs_ref[pl.ds(2 * j + 1, 1), :] = jnp.reshape(am1, (1, TS))
        return carry

    lax.fori_loop(0, K // 2, body2, 0)

    vals = vals_ref[...]
    idxs = idxs_ref[...]
    base = idxs[0:1, :]
    e0 = jnp.where(vals[:NS0] <= R0SQ, idxs[:NS0], base)
    e1 = jnp.where(vals <= R1SQ, idxs, base)
    idx_ref[0, 0] = jnp.concatenate([e0, e1], axis=0) + b * N


def _ball_query(posT, new_pos):
    return pl.pallas_call(
        _topk_body,
        grid=(B, NBLK),
        in_specs=[
            pl.BlockSpec((1, 3, N), lambda b, s: (b, 0, 0)),
            pl.BlockSpec((1, TS, 3), lambda b, s: (b, s, 0)),
        ],
        out_specs=pl.BlockSpec((1, 1, NIDX, TS), lambda b, s: (b, s, 0, 0)),
        out_shape=jax.ShapeDtypeStruct((B, NBLK, NIDX, TS), jnp.int32),
        scratch_shapes=[
            pltpu.VMEM((TS, N), jnp.float32),
            pltpu.VMEM((K, TS), jnp.float32),
            pltpu.VMEM((K, TS), jnp.int32),
        ],
    )(posT, new_pos)


def _mm_body(a_ref, w_ref, o_ref):
    ab = jnp.dot(a_ref[...], w_ref[...], preferred_element_type=jnp.float32)
    o_ref[...] = jnp.concatenate(
        [ab, jnp.zeros((TM, 128 - OUT), jnp.float32)], axis=1)


TM = 1024


def _proj(a_all, w_all):
    rows = a_all.shape[0]
    return pl.pallas_call(
        _mm_body,
        grid=(rows // TM,),
        in_specs=[
            pl.BlockSpec((TM, F), lambda i: (i, 0)),
            pl.BlockSpec((F, OUT), lambda i: (0, 0)),
        ],
        out_specs=pl.BlockSpec((TM, 128), lambda i: (i, 0)),
        out_shape=jax.ShapeDtypeStruct((rows, 128), jnp.float32),
    )(a_all, w_all)


def _sc_gather_max(proj, idxf):
    mesh = plsc.VectorSubcoreMesh(core_axis_name="c", subcore_axis_name="s")

    @functools.partial(
        pl.kernel,
        mesh=mesh,
        out_type=jax.ShapeDtypeStruct((B * S, 2 * OUT), jnp.float32),
        scratch_types=[
            pltpu.VMEM((GI,), jnp.int32),
            pltpu.VMEM((GI,), jnp.int32),
            pltpu.VMEM((GI, 128), jnp.float32),
            pltpu.VMEM((GI, 128), jnp.float32),
            pltpu.VMEM((CH, 128), jnp.float32),
            pltpu.VMEM((CH, 128), jnp.float32),
            pltpu.VMEM((CH, 2 * OUT), jnp.float32),
            pltpu.SemaphoreType.DMA,
            pltpu.SemaphoreType.DMA,
        ],
    )
    def k(p_hbm, idx_hbm, out_hbm, idx_v0, idx_v1, rows_v0, rows_v1,
          c_v0, c_v1, o_v, sem0, sem1):
        wid = lax.axis_index("c") * NS_SC + lax.axis_index("s")
        base0 = wid * CPW
        sems = (sem0, sem1)
        idxs_v = (idx_v0, idx_v1)
        rows_vs = (rows_v0, rows_v1)
        c_vs = (c_v0, c_v1)

        def fire(ci, buf):
            cb = base0 + ci * CH
            sem = sems[buf]
            idx_v, rows_v, c_v = idxs_v[buf], rows_vs[buf], c_vs[buf]
            pltpu.sync_copy(idx_hbm.at[pl.ds(cb * NIDX, GI)], idx_v)
            pltpu.async_copy(p_hbm.at[idx_v.at[pl.ds(0, GH)]],
                             rows_v.at[pl.ds(0, GH)], sem)
            pltpu.async_copy(p_hbm.at[idx_v.at[pl.ds(GH, GH)]],
                             rows_v.at[pl.ds(GH, GH)], sem)
            pltpu.async_copy(p_hbm.at[pl.ds(B * N + cb, CH)], c_v, sem)

        def drain(buf):
            sem = sems[buf]
            pltpu.make_async_copy(p_hbm.at[pl.ds(0, GI)],
                                  rows_vs[buf], sem).wait()
            pltpu.make_async_copy(p_hbm.at[pl.ds(0, CH)],
                                  c_vs[buf], sem).wait()

        def compute(ci, buf):
            cb = base0 + ci * CH
            rows_v, c_v = rows_vs[buf], c_vs[buf]
            for cc in range(CH):
                r0 = cc * NIDX
                for col in range(OUT // LANES):
                    sl = pl.ds(col * LANES, LANES)
                    cvec = c_v[cc, sl]
                    m0 = rows_v[r0, sl]
                    for j in range(1, NS0):
                        m0 = jnp.maximum(m0, rows_v[r0 + j, sl])
                    o_v[cc, sl] = jnp.maximum(m0 - cvec, 0.0)
                    m1 = rows_v[r0 + NS0, sl]
                    for j in range(NS0 + 1, NIDX):
                        m1 = jnp.maximum(m1, rows_v[r0 + j, sl])
                    o_v[cc, pl.ds(OUT + col * LANES, LANES)] = (
                        jnp.maximum(m1 - cvec, 0.0))
            pltpu.sync_copy(o_v, out_hbm.at[pl.ds(cb, CH)])

        fire(0, 0)

        def body2(j, carry):
            i0 = 2 * j
            fire(i0 + 1, 1)
            drain(0)
            compute(i0, 0)

            @pl.when(i0 + 2 < NCHUNK)
            def _():
                fire(i0 + 2, 0)

            drain(1)
            compute(i0 + 1, 1)
            return carry

        lax.fori_loop(0, NCHUNK // 2, body2, 0)

    return k(proj, idxf)


def kernel(x, pos, W, b):
    idxc = jnp.arange(S, dtype=jnp.int32) * (N // S)
    new_pos = pos[:, idxc, :]
    posT = jnp.transpose(pos, (0, 2, 1))

    idx48 = _ball_query(posT, new_pos)
    idxf = jnp.transpose(idx48, (0, 1, 3, 2)).reshape(-1)

    zg = jnp.zeros((B, N, 1), jnp.float32)
    a_g = jnp.concatenate([pos, x, zg], axis=-1).reshape(B * N, F)
    zc = jnp.zeros((B, S, C), jnp.float32)
    oc = jnp.ones((B, S, 1), jnp.float32)
    a_c = jnp.concatenate([new_pos, zc, oc], axis=-1).reshape(B * S, F)
    a_all = jnp.concatenate([a_g, a_c], axis=0)
    w_all = jnp.concatenate([W, -b[None, :]], axis=0)

    proj = _proj(a_all, w_all)
    out = _sc_gather_max(proj, idxf)
    return out.reshape(B, S, 2 * OUT), new_pos
